# Initial kernel scaffold; baseline (speedup 1.0000x reference)
import jax
import jax.numpy as jnp
from jax.experimental import pallas as pl


def kernel(node_feat, edge_index, W1_base, W2_base, W3_base, W_cls_base, A1, B1, A2, B2, A3, B3, Ac, Bc, g1, be1, g2, be2, g3, be3):
    raise NotImplementedError("write your pallas kernel here")


# speedup vs baseline: 2.2715x; 2.2715x over previous
import functools

import jax
import jax.numpy as jnp
from jax import lax
from jax.experimental import pallas as pl
from jax.experimental.pallas import tpu as pltpu
from jax.experimental.pallas import tpu_sc as plsc

NC = 2
NS = 16
NW = NC * NS
LANES = 16
EPS = 1e-5
SCALING = 0.25

NPAD = 10240
RPT = NPAD // NW
CH = 80
BIG = 2000
STG = 128


def _tile_id():
    return lax.axis_index("s") * NC + lax.axis_index("c")


def _prefix16(mi):
    csum = mi
    iota = lax.iota(jnp.int32, LANES)
    zv = jnp.zeros((LANES,), jnp.int32)
    onev = jnp.full((LANES,), 1, jnp.int32)
    sh31 = jnp.full((LANES,), 31, jnp.int32)
    for dshift in (1, 2, 4, 8):
        dv = jnp.full((LANES,), dshift, jnp.int32)
        im = iota - dv
        perm = jnp.maximum(im, zv)
        mk = onev - lax.shift_right_logical(im, sh31)
        csum = csum + csum.at[perm].get(mode="promise_in_bounds") * mk
    return csum


def _make_index_kernel(e):
    nbig = e // BIG
    nin = BIG // LANES
    mesh = plsc.VectorSubcoreMesh(core_axis_name="c", subcore_axis_name="s")

    @functools.partial(
        pl.kernel,
        mesh=mesh,
        out_type=(
            jax.ShapeDtypeStruct((NW * e,), jnp.int32),
            jax.ShapeDtypeStruct((NW * e,), jnp.int32),
            jax.ShapeDtypeStruct((NW * 16,), jnp.int32),
            jax.ShapeDtypeStruct((NPAD * 16,), jnp.float32),
        ),
        scratch_types=[
            pltpu.VMEM((BIG,), jnp.int32),
            pltpu.VMEM((BIG,), jnp.int32),
            pltpu.VMEM((STG,), jnp.int32),
            pltpu.VMEM((STG,), jnp.int32),
            pltpu.VMEM((16,), jnp.int32),
            pltpu.VMEM((16,), jnp.int32),
            pltpu.VMEM((RPT * 16,), jnp.float32),
        ],
    )
    def index_kernel(src_hbm, dst_hbm, slist_hbm, dlist_hbm, cnt_hbm,
                     deg_hbm, sbig, dbig, sstage, dstage, lreg, crow, dacc):
        t = _tile_id()
        lo = t * RPT
        lreg[pl.ds(0, 16)] = jnp.full((LANES,), lo, jnp.int32)

        for z in range(STG // LANES):
            sstage[pl.ds(z * LANES, LANES)] = jnp.zeros((LANES,), jnp.int32)
            dstage[pl.ds(z * LANES, LANES)] = jnp.zeros((LANES,), jnp.int32)

        def zrow(i, carry):
            dacc[pl.ds(i * LANES, LANES)] = jnp.zeros((LANES,), jnp.float32)
            return carry

        lax.fori_loop(0, RPT, zrow, 0)

        def flush(listk):
            def bump(i, carry):
                r = dstage[pl.ds(i, 16)][0]
                v = dacc[pl.ds(r * LANES, LANES)]
                dacc[pl.ds(r * LANES, LANES)] = v + jnp.ones(
                    (LANES,), jnp.float32)
                return carry

            lax.fori_loop(0, CH, bump, 0)
            base = t * e + listk * CH
            pltpu.sync_copy(sstage.at[pl.ds(0, CH)],
                            slist_hbm.at[pl.ds(base, CH)])
            pltpu.sync_copy(dstage.at[pl.ds(0, CH)],
                            dlist_hbm.at[pl.ds(base, CH)])
            sv = sstage[pl.ds(CH, 16)]
            dv = dstage[pl.ds(CH, 16)]
            sstage[pl.ds(0, 16)] = sv
            dstage[pl.ds(0, 16)] = dv

        def big_step(g, carry):
            off, listk = carry
            pltpu.sync_copy(src_hbm.at[pl.ds(g * BIG, BIG)], sbig)
            pltpu.sync_copy(dst_hbm.at[pl.ds(g * BIG, BIG)], dbig)

            def chunk(q, carry2):
                off2, listk2 = carry2
                d16 = dbig[pl.ds(q * LANES, LANES)]
                s16 = sbig[pl.ds(q * LANES, LANES)]
                lo_v = lreg[pl.ds(0, 16)]
                a = d16 - lo_v
                b = jnp.full((LANES,), RPT - 1, jnp.int32) - a
                mi = (jnp.full((LANES,), 1, jnp.int32)
                      - lax.shift_right_logical(
                          jnp.bitwise_or(a, b),
                          jnp.full((LANES,), 31, jnp.int32)))
                csum = _prefix16(mi)
                nsel = csum[15]

                @pl.when(nsel > 0)
                def _():
                    keep = jnp.minimum(lax.iota(jnp.int32, LANES),
                                       jnp.full((LANES,), 1, jnp.int32))
                    put = jnp.full((LANES,), 1, jnp.int32) - keep
                    for k in range(LANES):
                        sel = mi[k]
                        slot = off2 + csum[k] - sel

                        @pl.when(sel == 1)
                        def _():
                            kv = jnp.full((LANES,), k, jnp.int32)
                            s_spl = s16.at[kv].get(mode="promise_in_bounds")
                            a_spl = a.at[kv].get(mode="promise_in_bounds")
                            w = sstage[pl.ds(slot, 16)]
                            sstage[pl.ds(slot, 16)] = w * keep + s_spl * put
                            w2 = dstage[pl.ds(slot, 16)]
                            dstage[pl.ds(slot, 16)] = w2 * keep + a_spl * put

                off2 = off2 + nsel

                def do_flush(args):
                    o, kk = args
                    flush(kk)
                    return o - CH, kk + 1

                off2, listk2 = lax.cond(off2 >= CH, do_flush,
                                        lambda args: args, (off2, listk2))
                return off2, listk2

            return lax.fori_loop(0, nin, chunk, (off, listk))

        off, listk = lax.fori_loop(0, nbig, big_step, (0, 0))

        def bump_tail(i, carry):
            r = dstage[pl.ds(i, 16)][0]
            v = dacc[pl.ds(r * LANES, LANES)]
            dacc[pl.ds(r * LANES, LANES)] = v + jnp.ones((LANES,), jnp.float32)
            return carry

        lax.fori_loop(0, off, bump_tail, 0)
        base = t * e + listk * CH
        pltpu.sync_copy(sstage.at[pl.ds(0, CH)], slist_hbm.at[pl.ds(base, CH)])
        pltpu.sync_copy(dstage.at[pl.ds(0, CH)], dlist_hbm.at[pl.ds(base, CH)])

        crow[pl.ds(0, 16)] = jnp.full((LANES,), listk * CH + off, jnp.int32)
        pltpu.sync_copy(crow, cnt_hbm.at[pl.ds(t * 16, 16)])
        pltpu.sync_copy(dacc, deg_hbm.at[pl.ds(t * RPT * 16, RPT * 16)])

    return index_kernel


def _make_agg_kernel(d, e):
    mesh = plsc.VectorSubcoreMesh(core_axis_name="c", subcore_axis_name="s")

    @functools.partial(
        pl.kernel,
        mesh=mesh,
        out_type=jax.ShapeDtypeStruct((NPAD * d,), jnp.float32),
        scratch_types=[
            pltpu.VMEM((CH,), jnp.int32),
            pltpu.VMEM((CH + 16,), jnp.int32),
            pltpu.VMEM((16,), jnp.int32),
            pltpu.VMEM((CH, d), jnp.float32),
            pltpu.VMEM((RPT * d,), jnp.float32),
            pltpu.SemaphoreType.DMA,
        ],
    )
    def agg_kernel(x_hbm, slist_hbm, dlist_hbm, cnt_hbm, out_hbm,
                   sbuf, dbuf, crow, rows, acc, gsem):
        t = _tile_id()

        def zrow(i, carry):
            acc[pl.ds(i * LANES, LANES)] = jnp.zeros((LANES,), jnp.float32)
            return carry

        lax.fori_loop(0, RPT * d // LANES, zrow, 0)
        pltpu.sync_copy(cnt_hbm.at[pl.ds(t * 16, 16)], crow)
        cnt = crow[pl.ds(0, 16)][0]
        nblk = (cnt + CH - 1) // CH

        def blk(k, carry):
            base = t * e + k * CH
            pltpu.sync_copy(slist_hbm.at[pl.ds(base, CH)], sbuf)
            pltpu.sync_copy(dlist_hbm.at[pl.ds(base, CH)],
                            dbuf.at[pl.ds(0, CH)])
            pltpu.async_copy(x_hbm.at[sbuf], rows, gsem).wait()
            nsel = jnp.minimum(cnt - k * CH, CH)

            def rmw(i, carry2):
                r = dbuf[pl.ds(i, 16)][0]
                rb = r * d
                for u in range(d // LANES):
                    v = acc[pl.ds(rb + u * LANES, LANES)]
                    acc[pl.ds(rb + u * LANES, LANES)] = v + rows[
                        i, pl.ds(u * LANES, LANES)]
                return carry2

            lax.fori_loop(0, nsel, rmw, 0)
            return carry

        lax.fori_loop(0, nblk, blk, 0)
        pltpu.sync_copy(acc, out_hbm.at[pl.ds(t * RPT * d, RPT * d)])

    return agg_kernel


def _make_edge_kernel(n, d, e):
    nch = (e // NW) // CH
    mesh = plsc.VectorSubcoreMesh(core_axis_name="c", subcore_axis_name="s")

    @functools.partial(
        pl.kernel,
        mesh=mesh,
        out_type=jax.ShapeDtypeStruct((e, d), jnp.float32),
        scratch_types=[
            pltpu.VMEM((CH,), jnp.int32),
            pltpu.VMEM((CH,), jnp.int32),
            pltpu.VMEM((CH, d), jnp.float32),
            pltpu.VMEM((CH, d), jnp.float32),
            pltpu.SemaphoreType.DMA,
            pltpu.SemaphoreType.DMA,
        ],
    )
    def edge_kernel(h_hbm, src_hbm, dst_hbm, out_hbm,
                    sidx_v, didx_v, rowa_v, rowb_v, sema, semb):
        wid = _tile_id()

        def step(j, carry):
            base = wid * (nch * CH) + j * CH
            pltpu.sync_copy(src_hbm.at[pl.ds(base, CH)], sidx_v)
            pltpu.sync_copy(dst_hbm.at[pl.ds(base, CH)], didx_v)
            cpa = pltpu.async_copy(h_hbm.at[sidx_v], rowa_v, sema)
            cpb = pltpu.async_copy(h_hbm.at[didx_v], rowb_v, semb)
            cpa.wait()
            cpb.wait()

            def mul_row(i, carry2):
                for u in range(d // LANES):
                    sl = pl.ds(u * LANES, LANES)
                    rowa_v[i, sl] = rowa_v[i, sl] * rowb_v[i, sl]
                return carry2

            lax.fori_loop(0, CH, mul_row, 0)
            pltpu.sync_copy(rowa_v, out_hbm.at[pl.ds(base, CH)])
            return carry

        lax.fori_loop(0, nch, step, 0)

    return edge_kernel


def _prep_body(degp_ref, nf_ref, dinv_ref, x1_ref):
    deg = degp_ref[:, 0:1] + 1.0
    dinv = lax.rsqrt(deg)
    dinv_ref[...] = jnp.broadcast_to(dinv, dinv_ref.shape)
    x1_ref[...] = nf_ref[...] * dinv


def _run_prep(degp, node_feat, blk=1000):
    n, d = node_feat.shape
    grid = n // blk
    return pl.pallas_call(
        _prep_body,
        grid=(grid,),
        in_specs=[
            pl.BlockSpec((blk, 16), lambda i: (i, 0)),
            pl.BlockSpec((blk, d), lambda i: (i, 0)),
        ],
        out_specs=[
            pl.BlockSpec((blk, 8), lambda i: (i, 0)),
            pl.BlockSpec((blk, d), lambda i: (i, 0)),
        ],
        out_shape=[
            jax.ShapeDtypeStruct((n, 8), jnp.float32),
            jax.ShapeDtypeStruct((n, d), jnp.float32),
        ],
    )(degp, node_feat)


def _layer_body(p_ref, h_ref, dinv_ref, w_ref, a_ref, b_ref, g_ref, be_ref,
                hn_ref, xn_ref=None, *, last):
    dinv = dinv_ref[:, 0:1]
    z = dinv * p_ref[...] + (dinv * dinv) * h_ref[...]
    w_eff = w_ref[...] + SCALING * lax.dot_general(
        a_ref[...], b_ref[...], (((1,), (1,)), ((), ())),
        preferred_element_type=jnp.float32)
    t = lax.dot_general(z, w_eff, (((1,), (1,)), ((), ())),
                        preferred_element_type=jnp.float32)
    mu = jnp.mean(t, axis=-1, keepdims=True)
    var = jnp.mean(jnp.square(t - mu), axis=-1, keepdims=True)
    t = (t - mu) * lax.rsqrt(var + EPS) * g_ref[...] + be_ref[...]
    h_next = jnp.maximum(t, 0.0)
    hn_ref[...] = h_next
    if not last:
        xn_ref[...] = h_next * dinv


def _run_layer(p, h, dinv, w, a, b, g, be, last, blk=1000):
    n, d = h.shape
    grid = n // blk
    out_shapes = [jax.ShapeDtypeStruct((n, d), jnp.float32)]
    out_specs = [pl.BlockSpec((blk, d), lambda i: (i, 0))]
    if not last:
        out_shapes.append(jax.ShapeDtypeStruct((n, d), jnp.float32))
        out_specs.append(pl.BlockSpec((blk, d), lambda i: (i, 0)))
    res = pl.pallas_call(
        functools.partial(_layer_body, last=last),
        grid=(grid,),
        in_specs=[
            pl.BlockSpec((blk, d), lambda i: (i, 0)),
            pl.BlockSpec((blk, d), lambda i: (i, 0)),
            pl.BlockSpec((blk, 8), lambda i: (i, 0)),
            pl.BlockSpec((d, d), lambda i: (0, 0)),
            pl.BlockSpec((d, 4), lambda i: (0, 0)),
            pl.BlockSpec((d, 4), lambda i: (0, 0)),
            pl.BlockSpec((d,), lambda i: (0,)),
            pl.BlockSpec((d,), lambda i: (0,)),
        ],
        out_specs=out_specs,
        out_shape=out_shapes,
    )(p, h, dinv, w, a, b, g, be)
    if last:
        return res[0], None
    return res


def _logits_body(he_ref, wc_ref, ac_ref, bc_ref, out_ref):
    wc_eff = wc_ref[...] + SCALING * lax.dot_general(
        ac_ref[...], bc_ref[...], (((1,), (1,)), ((), ())),
        preferred_element_type=jnp.float32)
    out_ref[...] = lax.dot_general(
        he_ref[...], wc_eff, (((1,), (1,)), ((), ())),
        preferred_element_type=jnp.float32)


def _run_logits(h_edge, wc, ac, bc, blk=8000):
    e, d = h_edge.shape
    c = wc.shape[0]
    grid = e // blk
    return pl.pallas_call(
        _logits_body,
        grid=(grid,),
        in_specs=[
            pl.BlockSpec((blk, d), lambda i: (i, 0)),
            pl.BlockSpec((c, d), lambda i: (0, 0)),
            pl.BlockSpec((c, 4), lambda i: (0, 0)),
            pl.BlockSpec((d, 4), lambda i: (0, 0)),
        ],
        out_specs=pl.BlockSpec((blk, c), lambda i: (i, 0)),
        out_shape=jax.ShapeDtypeStruct((e, c), jnp.float32),
    )(h_edge, wc, ac, bc)


def kernel(node_feat, edge_index, W1_base, W2_base, W3_base, W_cls_base,
           A1, B1, A2, B2, A3, B3, Ac, Bc, g1, be1, g2, be2, g3, be3):
    n, d = node_feat.shape
    e = edge_index.shape[1]
    src = edge_index[0]
    dst = edge_index[1]

    index_k = _make_index_kernel(e)
    agg_k = _make_agg_kernel(d, e)
    edge_k = _make_edge_kernel(n, d, e)

    slist, dlist, cnts, degf = index_k(src, dst)
    degp = degf.reshape(NPAD, 16)
    dinv, x = _run_prep(degp[:n], node_feat)

    h = node_feat
    for w, a, b, g, be, last in (
            (W1_base, A1, B1, g1, be1, False),
            (W2_base, A2, B2, g2, be2, False),
            (W3_base, A3, B3, g3, be3, True)):
        pf = agg_k(x, slist, dlist, cnts)
        p = pf.reshape(NPAD, d)
        h, x = _run_layer(p[:n], h, dinv, w, a, b, g, be, last)

    h_edge = edge_k(h, src, dst)
    logits = _run_logits(h_edge, W_cls_base, Ac, Bc)
    return (logits, h)
